# Initial kernel scaffold; baseline (speedup 1.0000x reference)
import jax
import jax.numpy as jnp
from jax.experimental import pallas as pl


def kernel(user_x, book_x, rates_edge_index, likes_edge_index, user_W, user_b, book_W, book_b):
    raise NotImplementedError("write your pallas kernel here")


# speedup vs baseline: 37.0953x; 37.0953x over previous
import functools

import jax
import jax.numpy as jnp
from jax import lax
from jax.experimental import pallas as pl
from jax.experimental.pallas import tpu as pltpu
from jax.experimental.pallas import tpu_sc as plsc

N = 5000
NPAD = 5120
CHUNK = NPAD // 16
F = 128
FH = 64
E = 320000
ET = E // 16
EB = 128
K = (ET + EB - 1) // EB
ETP = K * EB
ED_T = (4 * 160000) // 32

_mesh = plsc.VectorSubcoreMesh(
    core_axis_name="c", subcore_axis_name="s", num_cores=2, num_subcores=16)

_f32 = jnp.float32
_i32 = jnp.int32


def _deg_body(didx_hbm, out_hbm, idx_v, hist_v):
    c = lax.axis_index("c")
    s = lax.axis_index("s")
    wid = c * 16 + s
    pltpu.sync_copy(didx_hbm.at[c, s], idx_v)

    zero16 = jnp.zeros((16,), _f32)
    ones16 = jnp.ones((16,), _f32)
    lane = lax.iota(_i32, 16)

    def _zero(i, carry):
        hist_v[pl.ds(i * 16, 16)] = zero16
        return carry
    lax.fori_loop(0, NPAD, _zero, 0)

    def _scat(e, carry):
        ii = idx_v[pl.ds(e * 16, 16)]
        flat = ii + lane * NPAD
        plsc.addupdate_scatter(hist_v, [flat], ones16)
        return carry
    lax.fori_loop(0, ED_T // 16, _scat, 0)

    def _red(v, carry):
        acc = hist_v[pl.ds(v * 16, 16)]
        for b in range(1, 16):
            acc = acc + hist_v[pl.ds(b * NPAD + v * 16, 16)]
        hist_v[pl.ds(v * 16, 16)] = acc
        return carry
    lax.fori_loop(0, NPAD // 16, _red, 0)

    pltpu.sync_copy(hist_v.at[pl.ds(0, NPAD)], out_hbm.at[pl.ds(wid * NPAD, NPAD)])


_deg_kernel = pl.kernel(
    _deg_body,
    out_type=jax.ShapeDtypeStruct((32 * NPAD,), _f32),
    mesh=_mesh,
    compiler_params=pltpu.CompilerParams(needs_layout_passes=False),
    scratch_types=[
        pltpu.VMEM((ED_T,), _i32),
        pltpu.VMEM((16 * NPAD,), _f32),
    ],
)


def _prep_body(ux, bx, uw, ub, bw, bb, degp, fb_o, z0_o, x0_o, db_o):
    x0 = jnp.dot(ux[...], uw[...], preferred_element_type=_f32) + ub[...][None, :]
    fb_o[...] = (jnp.dot(bx[...], bw[...], preferred_element_type=_f32)
                 + bb[...][None, :]) * (1.0 / 3.0)
    deg = jnp.sum(degp[...].reshape(32, NPAD), axis=0)
    dis = jnp.where(deg > 0, 1.0 / jnp.sqrt(deg), 0.0)
    x0p = jnp.concatenate([x0, jnp.zeros((NPAD - N, F), _f32)], axis=0)
    z0p = dis[:, None] * x0p
    z0_o[...] = jnp.concatenate([z0p[:, :FH], z0p[:, FH:]], axis=0)
    x0_o[...] = jnp.concatenate([x0p[:, :FH], x0p[:, FH:]], axis=0)
    db_o[...] = jnp.broadcast_to(dis[:, None], (NPAD, FH))


_prep_call = pl.pallas_call(
    _prep_body,
    out_shape=[
        jax.ShapeDtypeStruct((N, F), _f32),
        jax.ShapeDtypeStruct((2 * NPAD, FH), _f32),
        jax.ShapeDtypeStruct((2 * NPAD, FH), _f32),
        jax.ShapeDtypeStruct((NPAD, FH), _f32),
    ],
)


def _mp_body(z0f, x0f, disb, rowi, coli, fout, z1out,
             rvm, cvm, g0, nA, db, fk, acc):
    c = lax.axis_index("c")
    s = lax.axis_index("s")
    nbase = s * CHUNK
    coff = c * NPAD

    def _fill_zero(ref2d):
        zero16 = jnp.zeros((16,), _f32)
        def body(r, carry):
            for k in range(FH // 16):
                ref2d[r, pl.ds(k * 16, 16)] = zero16
            return carry
        lax.fori_loop(0, CHUNK, body, 0)

    pltpu.sync_copy(rowi.at[c, s], rvm)
    pltpu.sync_copy(coli.at[s], cvm)
    pltpu.sync_copy(x0f.at[pl.ds(coff + nbase, CHUNK)], fk)
    pltpu.sync_copy(disb.at[pl.ds(nbase, CHUNK)], db)
    _fill_zero(nA)
    pltpu.sync_copy(nA, acc.at[pl.ds(nbase, CHUNK)])
    plsc.subcore_barrier()

    def _edge_pass(table_ref):
        def body(j, carry):
            pltpu.sync_copy(table_ref.at[rvm.at[j]], g0)
            pltpu.sync_copy(g0, acc.at[cvm.at[j]], add=True)
            return carry
        lax.fori_loop(0, K, body, 0)

    _edge_pass(z0f)
    plsc.subcore_barrier()

    pltpu.sync_copy(acc.at[pl.ds(nbase, CHUNK)], nA)
    def _node1(r, carry):
        for k in range(FH // 16):
            sl = pl.ds(k * 16, 16)
            a = nA[r, sl]
            d = db[r, sl]
            u = a * d
            fk[r, sl] = fk[r, sl] + u
            nA[r, sl] = u * d
        return carry
    lax.fori_loop(0, CHUNK, _node1, 0)
    pltpu.sync_copy(nA, z1out.at[pl.ds(coff + nbase, CHUNK)])
    _fill_zero(nA)
    pltpu.sync_copy(nA, acc.at[pl.ds(nbase, CHUNK)])
    plsc.subcore_barrier()

    _edge_pass(z1out)
    plsc.subcore_barrier()

    pltpu.sync_copy(acc.at[pl.ds(nbase, CHUNK)], nA)
    def _node2(r, carry):
        for k in range(FH // 16):
            sl = pl.ds(k * 16, 16)
            nA[r, sl] = (fk[r, sl] + nA[r, sl] * db[r, sl]) * (1.0 / 3.0)
        return carry
    lax.fori_loop(0, CHUNK, _node2, 0)
    pltpu.sync_copy(nA, fout.at[pl.ds(coff + nbase, CHUNK)])


_mp_kernel = pl.kernel(
    _mp_body,
    out_type=(
        jax.ShapeDtypeStruct((2 * NPAD, FH), _f32),
        jax.ShapeDtypeStruct((2 * NPAD, FH), _f32),
    ),
    mesh=_mesh,
    compiler_params=pltpu.CompilerParams(
        needs_layout_passes=False, use_tc_tiling_on_sc=False),
    scratch_types=[
        pltpu.VMEM((K, EB), _i32),
        pltpu.VMEM((K, EB), _i32),
        pltpu.VMEM((EB, FH), _f32),
        pltpu.VMEM((CHUNK, FH), _f32),
        pltpu.VMEM((CHUNK, FH), _f32),
        pltpu.VMEM((CHUNK, FH), _f32),
        pltpu.VMEM_SHARED((NPAD, FH), _f32),
    ],
)


def kernel(user_x, book_x, rates_edge_index, likes_edge_index,
           user_W, user_b, book_W, book_b):
    rates_edge_index = rates_edge_index.astype(_i32)
    likes_edge_index = likes_edge_index.astype(_i32)

    didx = jnp.concatenate([
        rates_edge_index[1], likes_edge_index[1],
        rates_edge_index[0], likes_edge_index[0],
    ]).reshape(2, 16, ED_T)

    row = jnp.concatenate([rates_edge_index[0], likes_edge_index[0]])
    col = jnp.concatenate([rates_edge_index[1], likes_edge_index[1]])
    pad = jnp.full((16 * ETP - E,), N, _i32)
    rowp = jnp.concatenate([row, pad]).reshape(16, K, EB)
    colp = jnp.concatenate([col, pad]).reshape(16, K, EB)
    rowi = jnp.stack([rowp, rowp + NPAD])

    deg_parts = _deg_kernel(didx)
    final_book, z0f, x0f, db = _prep_call(
        user_x, book_x, user_W, user_b, book_W, book_b, deg_parts)
    fout, _z1 = _mp_kernel(z0f, x0f, db, rowi, colp)

    final_user = (fout.reshape(2, NPAD, FH)[:, :N, :]
                  .transpose(1, 0, 2).reshape(N, F))
    return (final_user, final_book)
